# Initial kernel scaffold; baseline (speedup 1.0000x reference)
import jax
import jax.numpy as jnp
from jax.experimental import pallas as pl


def kernel(hot_data, cold_data, access_counter, indices):
    raise NotImplementedError("write your pallas kernel here")


# speedup vs baseline: 1.5976x; 1.5976x over previous
import functools

import jax
import jax.numpy as jnp
from jax import lax
from jax.experimental import pallas as pl
from jax.experimental.pallas import tpu as pltpu
from jax.experimental.pallas import tpu_sc as plsc

HOT_N, COLD_N, D, B = 1000, 10000, 256, 4096
OUT_N = HOT_N + B + COLD_N
NW = 32
L = 16
HOT_W = 32
COLD_S = 312
COLD_W = 352
CBLK = 88
B_PER_W = B // NW


def _body(hot_hbm, cold_hbm, ac_hbm, idx_hbm, out_hbm,
          idx_v, ac_v, hotbuf, rows_v, mask_v, flag_v, cbuf, sem):
    wid = lax.axis_index("s") * 2 + lax.axis_index("c")

    pltpu.sync_copy(idx_hbm, idx_v)
    pltpu.sync_copy(ac_hbm, ac_v)

    hstart = jnp.minimum(wid * HOT_W, HOT_N - HOT_W)
    pltpu.sync_copy(hot_hbm.at[pl.ds(hstart, HOT_W)], hotbuf)
    pltpu.sync_copy(hotbuf, out_hbm.at[pl.ds(hstart, HOT_W)])

    lo = jnp.minimum(wid * COLD_S, COLD_N - COLD_W)
    ones16 = jnp.ones((L,), jnp.float32)

    def zbody(j, c):
        flag_v[pl.ds(j * L, L)] = jnp.zeros((L,), jnp.float32)
        return c

    lax.fori_loop(0, COLD_W // L, zbody, 0)

    def fbody(j, c):
        idx16 = idx_v[pl.ds(j * L, L)]
        acg = plsc.load_gather(ac_v, [idx16])
        m = (acg > 5.0) & (idx16 >= lo) & (idx16 < lo + COLD_W)
        loc = jnp.where(m, idx16 - lo, 0)
        plsc.store_scatter(flag_v, [loc], ones16, mask=m)
        return c

    lax.fori_loop(0, B // L, fbody, 0)

    base2 = wid * B_PER_W
    pltpu.async_copy(cold_hbm.at[idx_v.at[pl.ds(base2, B_PER_W)]],
                     rows_v, sem).wait()

    def mbody(j, c):
        idx16 = idx_v[pl.ds(base2 + j * L, L)]
        acg = plsc.load_gather(ac_v, [idx16])
        mask_v[pl.ds(j * L, L)] = jnp.where(acg > 5.0, 1.0, 0.0)
        return c

    lax.fori_loop(0, B_PER_W // L, mbody, 0)

    def rbody(i, c):
        mv = plsc.load_gather(mask_v, [jnp.full((L,), i, jnp.int32)])
        for col in range(D // L):
            rows_v[i, pl.ds(col * L, L)] = rows_v[i, pl.ds(col * L, L)] * mv
        return c

    lax.fori_loop(0, B_PER_W, rbody, 0)
    pltpu.sync_copy(rows_v, out_hbm.at[pl.ds(HOT_N + base2, B_PER_W)])

    for b in range(COLD_W // CBLK):
        bstart = lo + b * CBLK
        pltpu.sync_copy(cold_hbm.at[pl.ds(bstart, CBLK)], cbuf)

        def cmul(i, c, _b=b):
            fv = plsc.load_gather(
                flag_v, [jnp.full((L,), _b * CBLK + i, jnp.int32)])
            keep = jnp.where(fv > 0.5, 0.0, 1.0)
            for col in range(D // L):
                cbuf[i, pl.ds(col * L, L)] = cbuf[i, pl.ds(col * L, L)] * keep
            return c

        lax.fori_loop(0, CBLK, cmul, 0)
        pltpu.sync_copy(cbuf, out_hbm.at[pl.ds(HOT_N + B + bstart, CBLK)])


@jax.jit
def kernel(hot_data, cold_data, access_counter, indices):
    kfn = pl.kernel(
        _body,
        out_type=jax.ShapeDtypeStruct((OUT_N, D), jnp.float32),
        scratch_types=[
            pltpu.VMEM((B,), jnp.int32),
            pltpu.VMEM((COLD_N,), jnp.float32),
            pltpu.VMEM((HOT_W, D), jnp.float32),
            pltpu.VMEM((B_PER_W, D), jnp.float32),
            pltpu.VMEM((B_PER_W,), jnp.float32),
            pltpu.VMEM((COLD_W,), jnp.float32),
            pltpu.VMEM((CBLK, D), jnp.float32),
            pltpu.SemaphoreType.DMA,
        ],
        mesh=plsc.VectorSubcoreMesh(core_axis_name="c", subcore_axis_name="s"),
        compiler_params=pltpu.CompilerParams(needs_layout_passes=False),
    )
    return kfn(hot_data, cold_data, access_counter, indices)


# speedup vs baseline: 1.8428x; 1.1535x over previous
import jax
import jax.numpy as jnp
from jax import lax
from jax.experimental import pallas as pl
from jax.experimental.pallas import tpu as pltpu
from jax.experimental.pallas import tpu_sc as plsc

HOT_N, COLD_N, D, B = 1000, 10000, 256, 4096
OUT_N = HOT_N + B + COLD_N
NW = 32
L = 16
HOT_W = 32
COLD_S = 312
COLD_W = 352
NBLK = 4
CBLK = COLD_W // NBLK
B_PER_W = B // NW


def _zero_row(buf, i):
    z = jnp.zeros((L,), jnp.float32)
    for col in range(D // L):
        buf[i, pl.ds(col * L, L)] = z


def _body(hot_hbm, cold_hbm, ac_hbm, idx_hbm, out_hbm,
          idx_v, ac_v, hotbuf, rows_v, flag_v, mrow_v, cbuf0, cbuf1,
          sem_in, sem_hot, sem_g, sem_r2, sem_l0, sem_l1, sem_s0, sem_s1):
    wid = lax.axis_index("s") * 2 + lax.axis_index("c")

    cp_idx = pltpu.async_copy(idx_hbm, idx_v, sem_in)
    cp_ac = pltpu.async_copy(ac_hbm, ac_v, sem_in)
    hstart = jnp.minimum(wid * HOT_W, HOT_N - HOT_W)
    cp_hl = pltpu.async_copy(hot_hbm.at[pl.ds(hstart, HOT_W)], hotbuf, sem_hot)

    cp_idx.wait()
    cp_ac.wait()

    base2 = wid * B_PER_W
    cp_g = pltpu.async_copy(cold_hbm.at[idx_v.at[pl.ds(base2, B_PER_W)]],
                            rows_v, sem_g)
    lo = jnp.minimum(wid * COLD_S, COLD_N - COLD_W)
    loads = [None] * NBLK
    bufs = [cbuf0, cbuf1]
    lsems = [sem_l0, sem_l1]
    ssems = [sem_s0, sem_s1]
    loads[0] = pltpu.async_copy(cold_hbm.at[pl.ds(lo, CBLK)], cbuf0, sem_l0)
    loads[1] = pltpu.async_copy(cold_hbm.at[pl.ds(lo + CBLK, CBLK)],
                                cbuf1, sem_l1)

    cp_hl.wait()
    cp_hs = pltpu.async_copy(hotbuf, out_hbm.at[pl.ds(hstart, HOT_W)], sem_hot)

    def zbody(j, c):
        flag_v[pl.ds(j * L, L)] = jnp.zeros((L,), jnp.int32)
        return c

    lax.fori_loop(0, COLD_W // L + 1, zbody, 0)

    ones_i = jnp.ones((L,), jnp.int32)

    def fbody(j, c):
        idx16 = idx_v[pl.ds(j * L, L)]
        acg = plsc.load_gather(ac_v, [idx16])
        m = (acg > 5.0) & (idx16 >= lo) & (idx16 < lo + COLD_W)
        loc = jnp.where(m, idx16 - lo, 0)
        plsc.store_scatter(flag_v, [loc], ones_i, mask=m)
        return c

    lax.fori_loop(0, B // L, fbody, 0)

    def mbody(j, c):
        idx16 = idx_v[pl.ds(base2 + j * L, L)]
        acg = plsc.load_gather(ac_v, [idx16])
        mrow_v[pl.ds(j * L, L)] = jnp.where(acg > 5.0, 0, 1).astype(jnp.int32)
        return c

    lax.fori_loop(0, B_PER_W // L, mbody, 0)

    cp_g.wait()

    def r2body(i, c):
        @pl.when(mrow_v[pl.ds(i, L)][0] > 0)
        def _():
            _zero_row(rows_v, i)
        return c

    lax.fori_loop(0, B_PER_W, r2body, 0)
    cp_r2 = pltpu.async_copy(rows_v, out_hbm.at[pl.ds(HOT_N + base2, B_PER_W)],
                             sem_r2)

    stores = [None] * NBLK
    for b in range(NBLK):
        buf = bufs[b % 2]
        loads[b].wait()

        def cbody(i, c, _b=b, _buf=buf):
            @pl.when(flag_v[pl.ds(_b * CBLK + i, L)][0] > 0)
            def _():
                _zero_row(_buf, i)
            return c

        lax.fori_loop(0, CBLK, cbody, 0)
        stores[b] = pltpu.async_copy(
            buf, out_hbm.at[pl.ds(HOT_N + B + lo + b * CBLK, CBLK)],
            ssems[b % 2])
        if b + 2 < NBLK:
            stores[b].wait()
            loads[b + 2] = pltpu.async_copy(
                cold_hbm.at[pl.ds(lo + (b + 2) * CBLK, CBLK)],
                bufs[b % 2], lsems[b % 2])

    stores[NBLK - 2].wait()
    stores[NBLK - 1].wait()
    cp_r2.wait()
    cp_hs.wait()


@jax.jit
def kernel(hot_data, cold_data, access_counter, indices):
    kfn = pl.kernel(
        _body,
        out_type=jax.ShapeDtypeStruct((OUT_N, D), jnp.float32),
        scratch_types=[
            pltpu.VMEM((B,), jnp.int32),
            pltpu.VMEM((COLD_N,), jnp.float32),
            pltpu.VMEM((HOT_W, D), jnp.float32),
            pltpu.VMEM((B_PER_W, D), jnp.float32),
            pltpu.VMEM((COLD_W + L,), jnp.int32),
            pltpu.VMEM((B_PER_W + L,), jnp.int32),
            pltpu.VMEM((CBLK, D), jnp.float32),
            pltpu.VMEM((CBLK, D), jnp.float32),
            pltpu.SemaphoreType.DMA,
            pltpu.SemaphoreType.DMA,
            pltpu.SemaphoreType.DMA,
            pltpu.SemaphoreType.DMA,
            pltpu.SemaphoreType.DMA,
            pltpu.SemaphoreType.DMA,
            pltpu.SemaphoreType.DMA,
            pltpu.SemaphoreType.DMA,
        ],
        mesh=plsc.VectorSubcoreMesh(core_axis_name="c", subcore_axis_name="s"),
        compiler_params=pltpu.CompilerParams(needs_layout_passes=False),
    )
    return kfn(hot_data, cold_data, access_counter, indices)
